# baseline (speedup 1.0000x reference)
import functools

import jax
import jax.numpy as jnp
from jax import lax
from jax.experimental import pallas as pl
from jax.experimental.pallas import tpu as pltpu
from jax.experimental.pallas import tpu_sc as plsc

BATCH = 16384
EMBED_DIM = 64
PAIR_DIM = 2 * EMBED_DIM
OUT_DIM = 3 * EMBED_DIM + 1
MEMBERSHIP_VOCAB = 5
ID_VOCAB = 1000000
ZIP_VOCAB = 100000
NUM_CORES = 2
NUM_SUBCORES = 16
NUM_WORKERS = NUM_CORES * NUM_SUBCORES
B_PER_W = BATCH // NUM_WORKERS
CHUNK = 128
NCHUNK = B_PER_W // CHUNK
LANES = 16
AGE_MEAN = 35.0
AGE_STD = 14.0

_mesh = plsc.VectorSubcoreMesh(
    core_axis_name="c", subcore_axis_name="s",
    num_cores=NUM_CORES, num_subcores=NUM_SUBCORES,
)


@functools.partial(
    pl.kernel,
    out_type=jax.ShapeDtypeStruct((BATCH, OUT_DIM), jnp.float32),
    mesh=_mesh,
    scratch_types=[
        pltpu.VMEM((CHUNK,), jnp.int32),
        pltpu.VMEM((CHUNK,), jnp.int32),
        pltpu.VMEM((CHUNK,), jnp.int32),
        pltpu.VMEM((CHUNK,), jnp.int32),
        pltpu.VMEM((CHUNK,), jnp.int32),
        pltpu.VMEM((CHUNK,), jnp.float32),
        pltpu.VMEM((CHUNK, PAIR_DIM), jnp.float32),
        pltpu.VMEM((CHUNK, PAIR_DIM), jnp.float32),
        pltpu.VMEM((MEMBERSHIP_VOCAB, EMBED_DIM), jnp.float32),
        pltpu.VMEM((CHUNK, OUT_DIM), jnp.float32),
        pltpu.SemaphoreType.DMA,
        pltpu.SemaphoreType.DMA,
    ],
)
def _embed_kernel(cust_hbm, club_hbm, post_hbm, age_hbm,
                  id2_tab, mem_tab, zip2_tab, out_hbm,
                  ids_v, post_v, club_v, idh1_v, idh3_v, age_v,
                  buf1_v, buf3_v, mem_v, asm_v, sem1, sem3):
    wid = lax.axis_index("s") * NUM_CORES + lax.axis_index("c")
    base = wid * B_PER_W
    inv_std = jnp.float32(1.0 / AGE_STD)

    pltpu.sync_copy(mem_tab, mem_v)

    def chunk_body(k, carry):
        off = base + k * CHUNK
        pltpu.sync_copy(cust_hbm.at[pl.ds(off, CHUNK)], ids_v)
        pltpu.sync_copy(post_hbm.at[pl.ds(off, CHUNK)], post_v)
        pltpu.sync_copy(club_hbm.at[pl.ds(off, CHUNK)], club_v)
        pltpu.sync_copy(age_hbm.at[pl.ds(off, CHUNK)], age_v)

        def half_body(g, carry2):
            s = pl.ds(g * LANES, LANES)
            idh1_v[s] = lax.shift_right_logical(ids_v[s], 1)
            idh3_v[s] = lax.shift_right_logical(post_v[s], 1)
            return carry2

        lax.fori_loop(0, CHUNK // LANES, half_body, 0)

        cp1 = pltpu.make_async_copy(id2_tab.at[idh1_v], buf1_v, sem1)
        cp3 = pltpu.make_async_copy(zip2_tab.at[idh3_v], buf3_v, sem3)
        cp1.start()
        cp3.start()
        cp1.wait()
        cp3.wait()

        def group_body(g, carry2):
            sg = pl.ds(g * LANES, LANES)
            a16 = (age_v[sg] - AGE_MEAN) * inv_std
            c16 = club_v[sg]
            p1 = (ids_v[sg] & 1) * EMBED_DIM
            p3 = (post_v[sg] & 1) * EMBED_DIM
            for i in range(LANES):
                r = g * LANES + i
                m = c16[i]
                o1 = p1[i]
                o3 = p3[i]
                asm_v[r, pl.ds(OUT_DIM - LANES, LANES)] = jnp.broadcast_to(
                    a16[i], (LANES,))
                for c in range(EMBED_DIM // LANES):
                    s = pl.ds(c * LANES, LANES)
                    asm_v[r, pl.ds(c * LANES, LANES)] = \
                        buf1_v[r, pl.ds(o1 + c * LANES, LANES)]
                    asm_v[r, pl.ds(EMBED_DIM + c * LANES, LANES)] = mem_v[m, s]
                    asm_v[r, pl.ds(2 * EMBED_DIM + c * LANES, LANES)] = \
                        buf3_v[r, pl.ds(o3 + c * LANES, LANES)]
            return carry2

        lax.fori_loop(0, CHUNK // LANES, group_body, 0)

        pltpu.sync_copy(asm_v, out_hbm.at[pl.ds(off, CHUNK)])
        return carry

    lax.fori_loop(0, NCHUNK, chunk_body, 0)


def kernel(customer_id, club_member_status, postal_code, age,
           id_table, membership_table, zip_table):
    return _embed_kernel(
        customer_id.astype(jnp.int32),
        club_member_status.astype(jnp.int32),
        postal_code.astype(jnp.int32),
        age.astype(jnp.float32),
        id_table.reshape(ID_VOCAB // 2, PAIR_DIM),
        membership_table,
        zip_table.reshape(ZIP_VOCAB // 2, PAIR_DIM),
    )


# speedup vs baseline: 2.2557x; 2.2557x over previous
import functools

import jax
import jax.numpy as jnp
from jax import lax
from jax.experimental import pallas as pl
from jax.experimental.pallas import tpu as pltpu
from jax.experimental.pallas import tpu_sc as plsc

BATCH = 16384
EMBED_DIM = 64
OUT_DIM = 3 * EMBED_DIM + 1
MEMBERSHIP_VOCAB = 5
NUM_CORES = 2
NUM_SUBCORES = 16
NUM_WORKERS = NUM_CORES * NUM_SUBCORES
B_PER_W = BATCH // NUM_WORKERS
CHUNK = 128
NCHUNK = B_PER_W // CHUNK
LANES = 16
AGE_MEAN = 35.0
AGE_STD = 14.0

_mesh = plsc.VectorSubcoreMesh(
    core_axis_name="c", subcore_axis_name="s",
    num_cores=NUM_CORES, num_subcores=NUM_SUBCORES,
)


@functools.partial(
    pl.kernel,
    out_type=jax.ShapeDtypeStruct((BATCH, OUT_DIM), jnp.float32),
    mesh=_mesh,
    scratch_types=[
        pltpu.VMEM((CHUNK,), jnp.int32),
        pltpu.VMEM((CHUNK,), jnp.int32),
        pltpu.VMEM((CHUNK,), jnp.int32),
        pltpu.VMEM((CHUNK,), jnp.float32),
        pltpu.VMEM((CHUNK, EMBED_DIM), jnp.float32),
        pltpu.VMEM((CHUNK, EMBED_DIM), jnp.float32),
        pltpu.VMEM((MEMBERSHIP_VOCAB, EMBED_DIM), jnp.float32),
        pltpu.VMEM((CHUNK, OUT_DIM), jnp.float32),
        pltpu.SemaphoreType.DMA,
        pltpu.SemaphoreType.DMA,
    ],
)
def _embed_kernel(cust_hbm, club_hbm, post_hbm, age_hbm,
                  id3_tab, mem_tab, zip3_tab, out_hbm,
                  ids_v, post_v, club_v, age_v,
                  buf1_v, buf3_v, mem_v, asm_v, sem1, sem3):
    wid = lax.axis_index("s") * NUM_CORES + lax.axis_index("c")
    base = wid * B_PER_W
    inv_std = jnp.float32(1.0 / AGE_STD)
    id_tab = id3_tab.reshape(2 * (id3_tab.shape[1]), EMBED_DIM)
    zip_tab = zip3_tab.reshape(2 * (zip3_tab.shape[1]), EMBED_DIM)

    pltpu.sync_copy(mem_tab, mem_v)

    def chunk_body(k, carry):
        off = base + k * CHUNK
        pltpu.sync_copy(cust_hbm.at[pl.ds(off, CHUNK)], ids_v)
        pltpu.sync_copy(post_hbm.at[pl.ds(off, CHUNK)], post_v)
        pltpu.sync_copy(club_hbm.at[pl.ds(off, CHUNK)], club_v)
        pltpu.sync_copy(age_hbm.at[pl.ds(off, CHUNK)], age_v)

        def fire(g, carry2):
            iv1 = ids_v[pl.ds(g * LANES, LANES)]
            iv3 = post_v[pl.ds(g * LANES, LANES)]
            for i in range(LANES):
                r = g * LANES + i
                pltpu.make_async_copy(
                    id_tab.at[iv1[i]], buf1_v.at[r], sem1).start()
                pltpu.make_async_copy(
                    zip_tab.at[iv3[i]], buf3_v.at[r], sem3).start()
            return carry2

        lax.fori_loop(0, CHUNK // LANES, fire, 0)

        def drain(r, carry2):
            pltpu.make_async_copy(
                id_tab.at[0], buf1_v.at[0], sem1).wait()
            pltpu.make_async_copy(
                zip_tab.at[0], buf3_v.at[0], sem3).wait()
            return carry2

        lax.fori_loop(0, CHUNK, drain, 0)

        def group_body(g, carry2):
            a16 = (age_v[pl.ds(g * LANES, LANES)] - AGE_MEAN) * inv_std
            c16 = club_v[pl.ds(g * LANES, LANES)]
            for i in range(LANES):
                r = g * LANES + i
                m = c16[i]
                asm_v[r, pl.ds(OUT_DIM - LANES, LANES)] = jnp.broadcast_to(
                    a16[i], (LANES,))
                for c in range(EMBED_DIM // LANES):
                    s = pl.ds(c * LANES, LANES)
                    asm_v[r, pl.ds(c * LANES, LANES)] = buf1_v[r, s]
                    asm_v[r, pl.ds(EMBED_DIM + c * LANES, LANES)] = mem_v[m, s]
                    asm_v[r, pl.ds(2 * EMBED_DIM + c * LANES, LANES)] = \
                        buf3_v[r, s]
            return carry2

        lax.fori_loop(0, CHUNK // LANES, group_body, 0)

        pltpu.sync_copy(asm_v, out_hbm.at[pl.ds(off, CHUNK)])
        return carry

    lax.fori_loop(0, NCHUNK, chunk_body, 0)


def kernel(customer_id, club_member_status, postal_code, age,
           id_table, membership_table, zip_table):
    return _embed_kernel(
        customer_id.astype(jnp.int32),
        club_member_status.astype(jnp.int32),
        postal_code.astype(jnp.int32),
        age.astype(jnp.float32),
        id_table.reshape(2, id_table.shape[0] // 2, EMBED_DIM),
        membership_table,
        zip_table.reshape(2, zip_table.shape[0] // 2, EMBED_DIM),
    )


# speedup vs baseline: 2.3011x; 1.0201x over previous
import functools

import jax
import jax.numpy as jnp
from jax import lax
from jax.experimental import pallas as pl
from jax.experimental.pallas import tpu as pltpu
from jax.experimental.pallas import tpu_sc as plsc

BATCH = 16384
EMBED_DIM = 64
OUT_DIM = 3 * EMBED_DIM + 1
MEMBERSHIP_VOCAB = 5
NUM_CORES = 2
NUM_SUBCORES = 16
NUM_WORKERS = NUM_CORES * NUM_SUBCORES
B_PER_W = BATCH // NUM_WORKERS
CHUNK = 128
NCHUNK = B_PER_W // CHUNK
LANES = 16
AGE_MEAN = 35.0
AGE_STD = 14.0

_mesh = plsc.VectorSubcoreMesh(
    core_axis_name="c", subcore_axis_name="s",
    num_cores=NUM_CORES, num_subcores=NUM_SUBCORES,
)


@functools.partial(
    pl.kernel,
    out_type=jax.ShapeDtypeStruct((BATCH, OUT_DIM), jnp.float32),
    mesh=_mesh,
    scratch_types=[
        pltpu.VMEM((B_PER_W,), jnp.int32),
        pltpu.VMEM((B_PER_W,), jnp.int32),
        pltpu.VMEM((B_PER_W,), jnp.int32),
        pltpu.VMEM((B_PER_W,), jnp.float32),
        pltpu.VMEM((CHUNK, EMBED_DIM), jnp.float32),
        pltpu.VMEM((CHUNK, EMBED_DIM), jnp.float32),
        pltpu.VMEM((CHUNK, EMBED_DIM), jnp.float32),
        pltpu.VMEM((CHUNK, EMBED_DIM), jnp.float32),
        pltpu.VMEM((MEMBERSHIP_VOCAB, EMBED_DIM), jnp.float32),
        pltpu.VMEM((CHUNK, OUT_DIM), jnp.float32),
        pltpu.SemaphoreType.DMA,
        pltpu.SemaphoreType.DMA,
        pltpu.SemaphoreType.DMA,
        pltpu.SemaphoreType.DMA,
        pltpu.SemaphoreType.DMA,
        pltpu.SemaphoreType.DMA,
    ],
)
def _embed_kernel(cust_hbm, club_hbm, post_hbm, age_hbm,
                  id3_tab, mem_tab, zip3_tab, out_hbm,
                  ids_v, post_v, club_v, age_v,
                  buf1a, buf1b, buf3a, buf3b, mem_v, asm_v,
                  sem1a, sem1b, sem3a, sem3b, sem_s, sem_o):
    wid = lax.axis_index("s") * NUM_CORES + lax.axis_index("c")
    base = wid * B_PER_W
    inv_std = jnp.float32(1.0 / AGE_STD)
    id_tab = id3_tab.reshape(2 * (id3_tab.shape[1]), EMBED_DIM)
    zip_tab = zip3_tab.reshape(2 * (zip3_tab.shape[1]), EMBED_DIM)

    bufs1 = (buf1a, buf1b)
    bufs3 = (buf3a, buf3b)
    sems1 = (sem1a, sem1b)
    sems3 = (sem3a, sem3b)

    st = [
        pltpu.make_async_copy(cust_hbm.at[pl.ds(base, B_PER_W)], ids_v, sem_s),
        pltpu.make_async_copy(post_hbm.at[pl.ds(base, B_PER_W)], post_v, sem_s),
        pltpu.make_async_copy(club_hbm.at[pl.ds(base, B_PER_W)], club_v, sem_s),
        pltpu.make_async_copy(age_hbm.at[pl.ds(base, B_PER_W)], age_v, sem_s),
        pltpu.make_async_copy(mem_tab, mem_v, sem_s),
    ]
    for cp in st:
        cp.start()
    for cp in st:
        cp.wait()

    def fire(k, b1, b3, s1, s3):
        def fire_g(g, carry):
            iv1 = ids_v[pl.ds(k * CHUNK + g * LANES, LANES)]
            iv3 = post_v[pl.ds(k * CHUNK + g * LANES, LANES)]
            for i in range(LANES):
                r = g * LANES + i
                pltpu.make_async_copy(id_tab.at[iv1[i]], b1.at[r], s1).start()
                pltpu.make_async_copy(zip_tab.at[iv3[i]], b3.at[r], s3).start()
            return carry

        lax.fori_loop(0, CHUNK // LANES, fire_g, 0)

    def drain(b1, b3, s1, s3):
        pltpu.make_async_copy(id_tab.at[pl.ds(0, CHUNK)], b1, s1).wait()
        pltpu.make_async_copy(zip_tab.at[pl.ds(0, CHUNK)], b3, s3).wait()

    def assemble(k, b1, b3):
        def group_body(g, carry):
            sg = pl.ds(k * CHUNK + g * LANES, LANES)
            a16 = (age_v[sg] - AGE_MEAN) * inv_std
            c16 = club_v[sg]
            for i in range(LANES):
                r = g * LANES + i
                m = c16[i]
                asm_v[r, pl.ds(OUT_DIM - LANES, LANES)] = jnp.broadcast_to(
                    a16[i], (LANES,))
                for c in range(EMBED_DIM // LANES):
                    s = pl.ds(c * LANES, LANES)
                    asm_v[r, pl.ds(c * LANES, LANES)] = b1[r, s]
                    asm_v[r, pl.ds(EMBED_DIM + c * LANES, LANES)] = mem_v[m, s]
                    asm_v[r, pl.ds(2 * EMBED_DIM + c * LANES, LANES)] = \
                        b3[r, s]
            return carry

        lax.fori_loop(0, CHUNK // LANES, group_body, 0)

    def out_cp(k):
        return pltpu.make_async_copy(
            asm_v, out_hbm.at[pl.ds(base + k * CHUNK, CHUNK)], sem_o)

    fire(0, bufs1[0], bufs3[0], sems1[0], sems3[0])
    for k in range(NCHUNK):
        p = k % 2
        q = (k + 1) % 2
        if k + 1 < NCHUNK:
            fire(k + 1, bufs1[q], bufs3[q], sems1[q], sems3[q])
        drain(bufs1[p], bufs3[p], sems1[p], sems3[p])
        if k > 0:
            out_cp(k - 1).wait()
        assemble(k, bufs1[p], bufs3[p])
        out_cp(k).start()
    out_cp(NCHUNK - 1).wait()


def kernel(customer_id, club_member_status, postal_code, age,
           id_table, membership_table, zip_table):
    return _embed_kernel(
        customer_id.astype(jnp.int32),
        club_member_status.astype(jnp.int32),
        postal_code.astype(jnp.int32),
        age.astype(jnp.float32),
        id_table.reshape(2, id_table.shape[0] // 2, EMBED_DIM),
        membership_table,
        zip_table.reshape(2, zip_table.shape[0] // 2, EMBED_DIM),
    )


# speedup vs baseline: 2.3749x; 1.0321x over previous
import functools

import jax
import jax.numpy as jnp
from jax import lax
from jax.experimental import pallas as pl
from jax.experimental.pallas import tpu as pltpu
from jax.experimental.pallas import tpu_sc as plsc

BATCH = 16384
EMBED_DIM = 64
OUT_DIM = 3 * EMBED_DIM + 1
MEMBERSHIP_VOCAB = 5
NUM_CORES = 2
NUM_SUBCORES = 16
NUM_WORKERS = NUM_CORES * NUM_SUBCORES
B_PER_W = BATCH // NUM_WORKERS
CHUNK = 128
NCHUNK = B_PER_W // CHUNK
LANES = 16
AGE_MEAN = 35.0
AGE_STD = 14.0

_mesh = plsc.VectorSubcoreMesh(
    core_axis_name="c", subcore_axis_name="s",
    num_cores=NUM_CORES, num_subcores=NUM_SUBCORES,
)


@functools.partial(
    pl.kernel,
    out_type=jax.ShapeDtypeStruct((BATCH, OUT_DIM), jnp.float32),
    mesh=_mesh,
    scratch_types=[
        pltpu.VMEM((B_PER_W,), jnp.int32),
        pltpu.VMEM((B_PER_W,), jnp.int32),
        pltpu.VMEM((B_PER_W,), jnp.int32),
        pltpu.VMEM((B_PER_W,), jnp.float32),
        pltpu.VMEM((CHUNK, EMBED_DIM), jnp.float32),
        pltpu.VMEM((CHUNK, EMBED_DIM), jnp.float32),
        pltpu.VMEM((CHUNK, EMBED_DIM), jnp.float32),
        pltpu.VMEM((CHUNK, EMBED_DIM), jnp.float32),
        pltpu.VMEM((MEMBERSHIP_VOCAB, EMBED_DIM), jnp.float32),
        pltpu.VMEM((CHUNK, OUT_DIM), jnp.float32),
        pltpu.SemaphoreType.DMA,
        pltpu.SemaphoreType.DMA,
        pltpu.SemaphoreType.DMA,
        pltpu.SemaphoreType.DMA,
        pltpu.SemaphoreType.DMA,
        pltpu.SemaphoreType.DMA,
    ],
)
def _embed_kernel(cust_hbm, club_hbm, post_hbm, age_hbm,
                  id3_tab, mem_tab, zip_tab, out_hbm,
                  ids_v, post_v, club_v, age_v,
                  buf1a, buf1b, buf3a, buf3b, mem_v, asm_v,
                  sem1a, sem1b, sem3a, sem3b, sem_s, sem_o):
    wid = lax.axis_index("s") * NUM_CORES + lax.axis_index("c")
    base = wid * B_PER_W
    inv_std = jnp.float32(1.0 / AGE_STD)
    id_tab = id3_tab.reshape(2 * (id3_tab.shape[1]), EMBED_DIM)

    bufs1 = (buf1a, buf1b)
    bufs3 = (buf3a, buf3b)
    sems1 = (sem1a, sem1b)
    sems3 = (sem3a, sem3b)

    st = [
        pltpu.make_async_copy(cust_hbm.at[pl.ds(base, B_PER_W)], ids_v, sem_s),
        pltpu.make_async_copy(post_hbm.at[pl.ds(base, B_PER_W)], post_v, sem_s),
        pltpu.make_async_copy(club_hbm.at[pl.ds(base, B_PER_W)], club_v, sem_s),
        pltpu.make_async_copy(age_hbm.at[pl.ds(base, B_PER_W)], age_v, sem_s),
        pltpu.make_async_copy(mem_tab, mem_v, sem_s),
    ]
    for cp in st:
        cp.start()
    for cp in st:
        cp.wait()

    def fire(k, b1, b3, s1, s3):
        def fire_g(g, carry):
            iv1 = ids_v[pl.ds(k * CHUNK + g * LANES, LANES)]
            iv3 = post_v[pl.ds(k * CHUNK + g * LANES, LANES)]
            for i in range(LANES):
                r = g * LANES + i
                pltpu.make_async_copy(id_tab.at[iv1[i]], b1.at[r], s1).start()
                pltpu.make_async_copy(zip_tab.at[iv3[i]], b3.at[r], s3).start()
            return carry

        lax.fori_loop(0, CHUNK // LANES, fire_g, 0)

    def drain(b1, b3, s1, s3):
        pltpu.make_async_copy(id_tab.at[pl.ds(0, CHUNK)], b1, s1).wait()
        pltpu.make_async_copy(zip_tab.at[pl.ds(0, CHUNK)], b3, s3).wait()

    def assemble(k, b1, b3):
        def group_body(g, carry):
            sg = pl.ds(k * CHUNK + g * LANES, LANES)
            a16 = (age_v[sg] - AGE_MEAN) * inv_std
            c16 = club_v[sg]
            for i in range(LANES):
                r = g * LANES + i
                m = c16[i]
                asm_v[r, pl.ds(OUT_DIM - LANES, LANES)] = jnp.broadcast_to(
                    a16[i], (LANES,))
                for c in range(EMBED_DIM // LANES):
                    s = pl.ds(c * LANES, LANES)
                    asm_v[r, pl.ds(c * LANES, LANES)] = b1[r, s]
                    asm_v[r, pl.ds(EMBED_DIM + c * LANES, LANES)] = mem_v[m, s]
                    asm_v[r, pl.ds(2 * EMBED_DIM + c * LANES, LANES)] = \
                        b3[r, s]
            return carry

        lax.fori_loop(0, CHUNK // LANES, group_body, 0)

    def out_cp(k):
        return pltpu.make_async_copy(
            asm_v, out_hbm.at[pl.ds(base + k * CHUNK, CHUNK)], sem_o)

    fire(0, bufs1[0], bufs3[0], sems1[0], sems3[0])
    for k in range(NCHUNK):
        p = k % 2
        q = (k + 1) % 2
        if k + 1 < NCHUNK:
            fire(k + 1, bufs1[q], bufs3[q], sems1[q], sems3[q])
        drain(bufs1[p], bufs3[p], sems1[p], sems3[p])
        if k > 0:
            out_cp(k - 1).wait()
        assemble(k, bufs1[p], bufs3[p])
        out_cp(k).start()
    out_cp(NCHUNK - 1).wait()


def kernel(customer_id, club_member_status, postal_code, age,
           id_table, membership_table, zip_table):
    return _embed_kernel(
        customer_id.astype(jnp.int32),
        club_member_status.astype(jnp.int32),
        postal_code.astype(jnp.int32),
        age.astype(jnp.float32),
        id_table.reshape(2, id_table.shape[0] // 2, EMBED_DIM),
        membership_table,
        zip_table,
    )
